# Initial kernel scaffold; baseline (speedup 1.0000x reference)
import jax
import jax.numpy as jnp
from jax.experimental import pallas as pl


def kernel(x, edge_index, batch, conv_W, conv_b, bn_w, bn_b, cls_W1, cls_b1, cls_W2, cls_b2, nov_W1, nov_b1, nov_W2, nov_b2):
    raise NotImplementedError("write your pallas kernel here")


# speedup vs baseline: 14.3320x; 14.3320x over previous
import functools

import jax
import jax.numpy as jnp
from jax import lax
from jax.experimental import pallas as pl
from jax.experimental.pallas import tpu as pltpu
from jax.experimental.pallas import tpu_sc as plsc

N, E, D, H, C, G, L = 10000, 320000, 128, 128, 16, 64, 4
NC, NS = 2, 16
NW = NC * NS
EPT = E // NW
CHUNK = 80
NCHUNK = EPT // CHUNK
NP = 10240
RPT = NP // NS
DEGW = 128
RB = 1000
NRB = N // RB


def _deg_body(dst_hbm, ones_hbm, zeros_hbm, out_hbm, dst_v, ones_v, deg_sh):
    cid = lax.axis_index("c")
    sid = lax.axis_index("s")
    wid = sid * NC + cid
    pltpu.sync_copy(dst_hbm.at[wid], dst_v)
    pltpu.sync_copy(ones_hbm.at[pl.ds(0, CHUNK)], ones_v)
    pltpu.sync_copy(zeros_hbm.at[pl.ds(sid * RPT, RPT)],
                    deg_sh.at[pl.ds(sid * RPT, RPT)])
    plsc.subcore_barrier()

    def step(j, carry):
        pltpu.sync_copy(ones_v, deg_sh.at[dst_v.at[j]], add=True)
        return carry

    lax.fori_loop(0, NCHUNK, step, 0)
    plsc.subcore_barrier()
    pltpu.sync_copy(deg_sh.at[pl.ds(sid * RPT, RPT)],
                    out_hbm.at[cid, pl.ds(sid * RPT, RPT)])


def _edge_body(m2_hbm, src_hbm, dst_hbm, zeros_hbm, out_hbm,
               src_v, dst_v, rows_v, agg_sh, sem):
    cid = lax.axis_index("c")
    sid = lax.axis_index("s")
    wid = sid * NC + cid
    pltpu.sync_copy(src_hbm.at[wid], src_v)
    pltpu.sync_copy(dst_hbm.at[wid], dst_v)
    pltpu.sync_copy(zeros_hbm.at[pl.ds(sid * RPT, RPT)],
                    agg_sh.at[pl.ds(sid * RPT, RPT)])
    plsc.subcore_barrier()

    def step(j, carry):
        pltpu.async_copy(m2_hbm.at[src_v.at[j]], rows_v, sem).wait()
        pltpu.sync_copy(rows_v, agg_sh.at[dst_v.at[j]], add=True)
        return carry

    lax.fori_loop(0, NCHUNK, step, 0)
    plsc.subcore_barrier()
    pltpu.sync_copy(agg_sh.at[pl.ds(sid * RPT, RPT)],
                    out_hbm.at[cid, pl.ds(sid * RPT, RPT)])


@functools.cache
def _sc_kernels():
    mesh = plsc.VectorSubcoreMesh(core_axis_name="c", subcore_axis_name="s")
    deg = pl.kernel(
        _deg_body,
        out_type=jax.ShapeDtypeStruct((NC, NP, DEGW), jnp.float32),
        mesh=mesh,
        scratch_types=[
            pltpu.VMEM((NCHUNK, CHUNK), jnp.int32),
            pltpu.VMEM((CHUNK, DEGW), jnp.float32),
            pltpu.VMEM_SHARED((NP, DEGW), jnp.float32),
        ],
    )
    edge = pl.kernel(
        _edge_body,
        out_type=jax.ShapeDtypeStruct((NC, NP, H), jnp.float32),
        mesh=mesh,
        scratch_types=[
            pltpu.VMEM((NCHUNK, CHUNK), jnp.int32),
            pltpu.VMEM((NCHUNK, CHUNK), jnp.int32),
            pltpu.VMEM((CHUNK, H), jnp.float32),
            pltpu.VMEM_SHARED((NP, H), jnp.float32),
            pltpu.SemaphoreType.DMA,
        ],
    )
    return deg, edge


def _deg_pass(dst3, ones_d, zeros_d):
    return _sc_kernels()[0](dst3, ones_d, zeros_d)


def _edge_pass(m2, src3, dst3, zeros_h):
    return _sc_kernels()[1](m2, src3, dst3, zeros_h)


def _k0_body(x_ref, w_ref, degp_ref, m2_ref, dis_ref):
    deg = 1.0 + degp_ref[0, :, 0:1] + degp_ref[1, :, 0:1]
    dis = lax.rsqrt(deg)
    dis_ref[...] = dis
    m2_ref[...] = jnp.dot(x_ref[...], w_ref[...],
                          preferred_element_type=jnp.float32) * dis


def _layer_body(s_ref, m2p_ref, dis_ref, k1_ref, k2_ref, w_ref, out_ref):
    dis = dis_ref[...]
    t = (s_ref[0] + s_ref[1] + m2p_ref[...]) * dis
    h = jnp.maximum(t * k1_ref[...] + k2_ref[...], 0.0)
    out_ref[...] = jnp.dot(h, w_ref[...],
                           preferred_element_type=jnp.float32) * dis


def _final_body(s_ref, m2p_ref, dis_ref, k1_ref, k2_ref, batch_ref,
                cw1_ref, cb1_ref, cw2_ref, cb2_ref,
                nw1_ref, nb1_ref, nw2_ref, nb2_ref,
                logits_ref, nov_ref, g_acc):
    i = pl.program_id(0)
    t = (s_ref[0] + s_ref[1] + m2p_ref[...]) * dis_ref[...]
    h = jnp.maximum(t * k1_ref[...] + k2_ref[...], 0.0)
    onehot = (batch_ref[...] ==
              lax.broadcasted_iota(jnp.int32, (RB, G), 1)).astype(jnp.float32)
    part = lax.dot_general(onehot, h, (((0,), (0,)), ((), ())),
                           preferred_element_type=jnp.float32)

    @pl.when(i == 0)
    def _():
        g_acc[...] = part

    @pl.when(i > 0)
    def _():
        g_acc[...] = g_acc[...] + part

    @pl.when(i == NRB - 1)
    def _():
        g = g_acc[...]
        t1 = jnp.maximum(jnp.dot(g, cw1_ref[...],
                                 preferred_element_type=jnp.float32)
                         + cb1_ref[...], 0.0)
        logits_ref[...] = jnp.dot(t1, cw2_ref[...],
                                  preferred_element_type=jnp.float32) + cb2_ref[...]
        t2 = jnp.maximum(jnp.dot(g, nw1_ref[...],
                                 preferred_element_type=jnp.float32)
                         + nb1_ref[...], 0.0)
        nov_ref[...] = jax.nn.sigmoid(
            jnp.dot(t2, nw2_ref[...], preferred_element_type=jnp.float32)
            + nb2_ref[...])


_ROW = lambda i: (i, 0)
_CONST2 = lambda i: (0, 0)


def _tc_first(x, w0, degp):
    return pl.pallas_call(
        _k0_body,
        grid=(NRB,),
        in_specs=[
            pl.BlockSpec((RB, D), _ROW),
            pl.BlockSpec((D, H), _CONST2),
            pl.BlockSpec((NC, RB, DEGW), lambda i: (0, i, 0)),
        ],
        out_specs=[pl.BlockSpec((RB, H), _ROW), pl.BlockSpec((RB, 1), _ROW)],
        out_shape=[jax.ShapeDtypeStruct((N, H), jnp.float32),
                   jax.ShapeDtypeStruct((N, 1), jnp.float32)],
    )(x, w0, degp)


def _tc_layer(s, m2p, dis2d, k1, k2, w):
    return pl.pallas_call(
        _layer_body,
        grid=(NRB,),
        in_specs=[
            pl.BlockSpec((NC, RB, H), lambda i: (0, i, 0)),
            pl.BlockSpec((RB, H), _ROW),
            pl.BlockSpec((RB, 1), _ROW),
            pl.BlockSpec((1, H), _CONST2),
            pl.BlockSpec((1, H), _CONST2),
            pl.BlockSpec((H, H), _CONST2),
        ],
        out_specs=pl.BlockSpec((RB, H), _ROW),
        out_shape=jax.ShapeDtypeStruct((N, H), jnp.float32),
    )(s, m2p, dis2d, k1, k2, w)


def _tc_final(s, m2p, dis2d, k1, k2, batch2,
              cw1, cb1, cw2, cb2, nw1, nb1, nw2, nb2):
    return pl.pallas_call(
        _final_body,
        grid=(NRB,),
        in_specs=[
            pl.BlockSpec((NC, RB, H), lambda i: (0, i, 0)),
            pl.BlockSpec((RB, H), _ROW),
            pl.BlockSpec((RB, 1), _ROW),
            pl.BlockSpec((1, H), _CONST2),
            pl.BlockSpec((1, H), _CONST2),
            pl.BlockSpec((RB, 1), _ROW),
            pl.BlockSpec((H, H), _CONST2),
            pl.BlockSpec((1, H), _CONST2),
            pl.BlockSpec((H, C), _CONST2),
            pl.BlockSpec((1, C), _CONST2),
            pl.BlockSpec((H, H), _CONST2),
            pl.BlockSpec((1, H), _CONST2),
            pl.BlockSpec((H, 1), _CONST2),
            pl.BlockSpec((1, 1), _CONST2),
        ],
        out_specs=[pl.BlockSpec((G, C), _CONST2),
                   pl.BlockSpec((G, 1), _CONST2)],
        out_shape=[jax.ShapeDtypeStruct((G, C), jnp.float32),
                   jax.ShapeDtypeStruct((G, 1), jnp.float32)],
        scratch_shapes=[pltpu.VMEM((G, H), jnp.float32)],
    )(s, m2p, dis2d, k1, k2, batch2, cw1, cb1, cw2, cb2, nw1, nb1, nw2, nb2)


def kernel(x, edge_index, batch, conv_W, conv_b, bn_w, bn_b,
           cls_W1, cls_b1, cls_W2, cls_b2, nov_W1, nov_b1, nov_W2, nov_b2):
    src3 = edge_index[0].reshape(NW, NCHUNK, CHUNK)
    dst3 = edge_index[1].reshape(NW, NCHUNK, CHUNK)
    zeros_h = jnp.zeros((NP, H), jnp.float32)
    zeros_d = jnp.zeros((NP, DEGW), jnp.float32)
    ones_d = jnp.ones((CHUNK, DEGW), jnp.float32)
    batch2 = batch.reshape(N, 1)

    k1 = bn_w * (1.0 / jnp.sqrt(jnp.float32(1.0 + 1e-5)))
    k2 = conv_b * k1 + bn_b

    degp = _deg_pass(dst3, ones_d, zeros_d)
    m2, dis2d = _tc_first(x, conv_W[0], degp)
    for l in range(L - 1):
        s = _edge_pass(m2, src3, dst3, zeros_h)
        m2 = _tc_layer(s, m2, dis2d,
                       k1[l].reshape(1, H), k2[l].reshape(1, H),
                       conv_W[l + 1])
    s = _edge_pass(m2, src3, dst3, zeros_h)
    logits, nov = _tc_final(
        s, m2, dis2d, k1[L - 1].reshape(1, H), k2[L - 1].reshape(1, H),
        batch2,
        cls_W1, cls_b1.reshape(1, H), cls_W2, cls_b2.reshape(1, C),
        nov_W1, nov_b1.reshape(1, H), nov_W2, nov_b2.reshape(1, 1))
    return logits, nov


# speedup vs baseline: 16.3702x; 1.1422x over previous
import functools

import jax
import jax.numpy as jnp
from jax import lax
from jax.experimental import pallas as pl
from jax.experimental.pallas import tpu as pltpu
from jax.experimental.pallas import tpu_sc as plsc

N, E, D, H, C, G, L = 10000, 320000, 128, 128, 16, 64, 4
NC, NS = 2, 16
NW = NC * NS
EPT = E // NW
CHUNK = 128
NCHUNK = 79
EPTP = NCHUNK * CHUNK
NPAD = EPTP - EPT
NP = 10240
RPT = NP // NS
DCH = 80
DNCH = EPT // DCH
DEGW = 128
RB = 1000
NRB = N // RB


def _deg_body(dst_hbm, ones_hbm, zeros_hbm, out_hbm, dst_v, ones_v, deg_sh):
    cid = lax.axis_index("c")
    sid = lax.axis_index("s")
    wid = sid * NC + cid
    pltpu.sync_copy(dst_hbm.at[wid], dst_v)
    pltpu.sync_copy(ones_hbm.at[pl.ds(0, DCH)], ones_v)
    pltpu.sync_copy(zeros_hbm.at[pl.ds(sid * RPT, RPT)],
                    deg_sh.at[pl.ds(sid * RPT, RPT)])
    plsc.subcore_barrier()

    def step(j, carry):
        pltpu.sync_copy(ones_v, deg_sh.at[dst_v.at[j]], add=True)
        return carry

    lax.fori_loop(0, DNCH, step, 0)
    plsc.subcore_barrier()
    pltpu.sync_copy(deg_sh.at[pl.ds(sid * RPT, RPT)],
                    out_hbm.at[cid, pl.ds(sid * RPT, RPT)])


def _edge_body(m2_hbm, src_hbm, dst_hbm, zeros_hbm, out_hbm,
               src_v, dst_v, rows_v, agg_sh, sem):
    cid = lax.axis_index("c")
    sid = lax.axis_index("s")
    wid = sid * NC + cid
    pltpu.sync_copy(src_hbm.at[wid], src_v)
    pltpu.sync_copy(dst_hbm.at[wid], dst_v)
    pltpu.sync_copy(zeros_hbm.at[pl.ds(sid * RPT, RPT)],
                    agg_sh.at[pl.ds(sid * RPT, RPT)])
    plsc.subcore_barrier()

    def step(j, carry):
        pltpu.async_copy(m2_hbm.at[src_v.at[j]], rows_v, sem).wait()
        pltpu.sync_copy(rows_v, agg_sh.at[dst_v.at[j]], add=True)
        return carry

    lax.fori_loop(0, NCHUNK, step, 0)
    plsc.subcore_barrier()
    pltpu.sync_copy(agg_sh.at[pl.ds(sid * RPT, RPT)],
                    out_hbm.at[cid, pl.ds(sid * RPT, RPT)])


@functools.cache
def _sc_kernels():
    mesh = plsc.VectorSubcoreMesh(core_axis_name="c", subcore_axis_name="s")
    deg = pl.kernel(
        _deg_body,
        out_type=jax.ShapeDtypeStruct((NC, NP, DEGW), jnp.float32),
        mesh=mesh,
        scratch_types=[
            pltpu.VMEM((DNCH, DCH), jnp.int32),
            pltpu.VMEM((DCH, DEGW), jnp.float32),
            pltpu.VMEM_SHARED((NP, DEGW), jnp.float32),
        ],
    )
    edge = pl.kernel(
        _edge_body,
        out_type=jax.ShapeDtypeStruct((NC, NP, H), jnp.float32),
        mesh=mesh,
        scratch_types=[
            pltpu.VMEM((NCHUNK, CHUNK), jnp.int32),
            pltpu.VMEM((NCHUNK, CHUNK), jnp.int32),
            pltpu.VMEM((CHUNK, H), jnp.float32),
            pltpu.VMEM_SHARED((NP, H), jnp.float32),
            pltpu.SemaphoreType.DMA,
        ],
    )
    return deg, edge


def _deg_pass(dst3, ones_d, zeros_d):
    return _sc_kernels()[0](dst3, ones_d, zeros_d)


def _edge_pass(m2, src3, dst3, zeros_h):
    return _sc_kernels()[1](m2, src3, dst3, zeros_h)


def _k0_body(x_ref, w_ref, degp_ref, m2_ref, dis_ref):
    deg = 1.0 + degp_ref[0, :, 0:1] + degp_ref[1, :, 0:1]
    dis = lax.rsqrt(deg)
    dis_ref[...] = dis
    m2_ref[...] = jnp.dot(x_ref[...], w_ref[...],
                          preferred_element_type=jnp.float32) * dis


def _layer_body(s_ref, m2p_ref, dis_ref, k1_ref, k2_ref, w_ref, out_ref):
    dis = dis_ref[...]
    t = (s_ref[0] + s_ref[1] + m2p_ref[...]) * dis
    h = jnp.maximum(t * k1_ref[...] + k2_ref[...], 0.0)
    out_ref[...] = jnp.dot(h, w_ref[...],
                           preferred_element_type=jnp.float32) * dis


def _final_body(s_ref, m2p_ref, dis_ref, k1_ref, k2_ref, batch_ref,
                cw1_ref, cb1_ref, cw2_ref, cb2_ref,
                nw1_ref, nb1_ref, nw2_ref, nb2_ref,
                logits_ref, nov_ref, g_acc):
    i = pl.program_id(0)
    t = (s_ref[0] + s_ref[1] + m2p_ref[...]) * dis_ref[...]
    h = jnp.maximum(t * k1_ref[...] + k2_ref[...], 0.0)
    onehot = (batch_ref[...] ==
              lax.broadcasted_iota(jnp.int32, (RB, G), 1)).astype(jnp.float32)
    part = lax.dot_general(onehot, h, (((0,), (0,)), ((), ())),
                           preferred_element_type=jnp.float32)

    @pl.when(i == 0)
    def _():
        g_acc[...] = part

    @pl.when(i > 0)
    def _():
        g_acc[...] = g_acc[...] + part

    @pl.when(i == NRB - 1)
    def _():
        g = g_acc[...]
        t1 = jnp.maximum(jnp.dot(g, cw1_ref[...],
                                 preferred_element_type=jnp.float32)
                         + cb1_ref[...], 0.0)
        logits_ref[...] = jnp.dot(t1, cw2_ref[...],
                                  preferred_element_type=jnp.float32) + cb2_ref[...]
        t2 = jnp.maximum(jnp.dot(g, nw1_ref[...],
                                 preferred_element_type=jnp.float32)
                         + nb1_ref[...], 0.0)
        nov_ref[...] = jax.nn.sigmoid(
            jnp.dot(t2, nw2_ref[...], preferred_element_type=jnp.float32)
            + nb2_ref[...])


_ROW = lambda i: (i, 0)
_CONST2 = lambda i: (0, 0)
_SPLIT = lambda i: (0, i, 0)


def _tc_first(x, w0, degp):
    return pl.pallas_call(
        _k0_body,
        grid=(NRB,),
        in_specs=[
            pl.BlockSpec((RB, D), _ROW),
            pl.BlockSpec((D, H), _CONST2),
            pl.BlockSpec((NC, RB, DEGW), _SPLIT),
        ],
        out_specs=[pl.BlockSpec((RB, H), _ROW), pl.BlockSpec((RB, 1), _ROW)],
        out_shape=[jax.ShapeDtypeStruct((N, H), jnp.float32),
                   jax.ShapeDtypeStruct((N, 1), jnp.float32)],
    )(x, w0, degp)


def _tc_layer(s, m2, dis2d, k1, k2, w):
    return pl.pallas_call(
        _layer_body,
        grid=(NRB,),
        in_specs=[
            pl.BlockSpec((NC, RB, H), _SPLIT),
            pl.BlockSpec((RB, H), _ROW),
            pl.BlockSpec((RB, 1), _ROW),
            pl.BlockSpec((1, H), _CONST2),
            pl.BlockSpec((1, H), _CONST2),
            pl.BlockSpec((H, H), _CONST2),
        ],
        out_specs=pl.BlockSpec((RB, H), _ROW),
        out_shape=jax.ShapeDtypeStruct((N, H), jnp.float32),
    )(s, m2, dis2d, k1, k2, w)


def _tc_final(s, m2, dis2d, k1, k2, batch2,
              cw1, cb1, cw2, cb2, nw1, nb1, nw2, nb2):
    return pl.pallas_call(
        _final_body,
        grid=(NRB,),
        in_specs=[
            pl.BlockSpec((NC, RB, H), _SPLIT),
            pl.BlockSpec((RB, H), _ROW),
            pl.BlockSpec((RB, 1), _ROW),
            pl.BlockSpec((1, H), _CONST2),
            pl.BlockSpec((1, H), _CONST2),
            pl.BlockSpec((RB, 1), _ROW),
            pl.BlockSpec((H, H), _CONST2),
            pl.BlockSpec((1, H), _CONST2),
            pl.BlockSpec((H, C), _CONST2),
            pl.BlockSpec((1, C), _CONST2),
            pl.BlockSpec((H, H), _CONST2),
            pl.BlockSpec((1, H), _CONST2),
            pl.BlockSpec((H, 1), _CONST2),
            pl.BlockSpec((1, 1), _CONST2),
        ],
        out_specs=[pl.BlockSpec((G, C), _CONST2),
                   pl.BlockSpec((G, 1), _CONST2)],
        out_shape=[jax.ShapeDtypeStruct((G, C), jnp.float32),
                   jax.ShapeDtypeStruct((G, 1), jnp.float32)],
        scratch_shapes=[pltpu.VMEM((G, H), jnp.float32)],
    )(s, m2, dis2d, k1, k2, batch2, cw1, cb1, cw2, cb2, nw1, nb1, nw2, nb2)


def kernel(x, edge_index, batch, conv_W, conv_b, bn_w, bn_b,
           cls_W1, cls_b1, cls_W2, cls_b2, nov_W1, nov_b1, nov_W2, nov_b2):
    src_t = edge_index[0].reshape(NW, EPT)
    dst_t = edge_index[1].reshape(NW, EPT)
    pad_i = jnp.arange(NPAD, dtype=jnp.int32)
    tile_i = jnp.arange(NW, dtype=jnp.int32)[:, None]
    src_pad = (tile_i * 311 + pad_i * 89) % N
    dst_pad = N + ((tile_i * 7 + pad_i) % (NP - N))
    src3 = jnp.concatenate([src_t, src_pad], axis=1).reshape(NW, NCHUNK, CHUNK)
    dst3 = jnp.concatenate([dst_t, dst_pad], axis=1).reshape(NW, NCHUNK, CHUNK)
    dstd = edge_index[1].reshape(NW, DNCH, DCH)
    zeros_h = jnp.zeros((NP, H), jnp.float32)
    zeros_d = jnp.zeros((NP, DEGW), jnp.float32)
    ones_d = jnp.ones((DCH, DEGW), jnp.float32)
    batch2 = batch.reshape(N, 1)

    k1 = bn_w * (1.0 / jnp.sqrt(jnp.float32(1.0 + 1e-5)))
    k2 = conv_b * k1 + bn_b

    degp = _deg_pass(dstd, ones_d, zeros_d)
    m2, dis2d = _tc_first(x, conv_W[0], degp)
    for l in range(L - 1):
        s = _edge_pass(m2, src3, dst3, zeros_h)
        m2 = _tc_layer(s, m2, dis2d,
                       k1[l].reshape(1, H), k2[l].reshape(1, H),
                       conv_W[l + 1])
    s = _edge_pass(m2, src3, dst3, zeros_h)
    logits, nov = _tc_final(
        s, m2, dis2d, k1[L - 1].reshape(1, H), k2[L - 1].reshape(1, H),
        batch2,
        cls_W1, cls_b1.reshape(1, H), cls_W2, cls_b2.reshape(1, C),
        nov_W1, nov_b1.reshape(1, H), nov_W2, nov_b2.reshape(1, 1))
    return logits, nov


# speedup vs baseline: 16.3942x; 1.0015x over previous
import functools

import jax
import jax.numpy as jnp
from jax import lax
from jax.experimental import pallas as pl
from jax.experimental.pallas import tpu as pltpu
from jax.experimental.pallas import tpu_sc as plsc

N, E, D, H, C, G, L = 10000, 320000, 128, 128, 16, 64, 4
NC, NS = 2, 16
NW = NC * NS
EPT = E // NW
CHUNK = 128
NCHUNK = 79
EPTP = NCHUNK * CHUNK
NPAD = EPTP - EPT
NP = 10240
RPT = NP // NS
DEGW = 128
RB = 1000
NRB = N // RB


def _deg_body(dst_hbm, ones_hbm, zeros_hbm, out_hbm, dst_v, ones_v, deg_sh):
    cid = lax.axis_index("c")
    sid = lax.axis_index("s")
    wid = sid * NC + cid
    pltpu.sync_copy(dst_hbm.at[wid], dst_v)
    pltpu.sync_copy(ones_hbm.at[pl.ds(0, CHUNK)], ones_v)
    pltpu.sync_copy(zeros_hbm.at[pl.ds(sid * RPT, RPT)],
                    deg_sh.at[pl.ds(sid * RPT, RPT)])
    plsc.subcore_barrier()

    def step(j, carry):
        pltpu.sync_copy(ones_v, deg_sh.at[dst_v.at[j]], add=True)
        return carry

    lax.fori_loop(0, NCHUNK, step, 0)
    plsc.subcore_barrier()
    pltpu.sync_copy(deg_sh.at[pl.ds(sid * RPT, RPT)],
                    out_hbm.at[cid, pl.ds(sid * RPT, RPT)])


def _edge_body(m2_hbm, src_hbm, dst_hbm, zeros_hbm, out_hbm,
               src_v, dst_v, rows_v, agg_sh, sem):
    cid = lax.axis_index("c")
    sid = lax.axis_index("s")
    wid = sid * NC + cid
    pltpu.sync_copy(src_hbm.at[wid], src_v)
    pltpu.sync_copy(dst_hbm.at[wid], dst_v)
    pltpu.sync_copy(zeros_hbm.at[pl.ds(sid * RPT, RPT)],
                    agg_sh.at[pl.ds(sid * RPT, RPT)])
    plsc.subcore_barrier()

    def step(j, carry):
        pltpu.async_copy(m2_hbm.at[src_v.at[j]], rows_v, sem).wait()
        pltpu.sync_copy(rows_v, agg_sh.at[dst_v.at[j]], add=True)
        return carry

    lax.fori_loop(0, NCHUNK, step, 0)
    plsc.subcore_barrier()
    pltpu.sync_copy(agg_sh.at[pl.ds(sid * RPT, RPT)],
                    out_hbm.at[cid, pl.ds(sid * RPT, RPT)])


@functools.cache
def _sc_kernels():
    mesh = plsc.VectorSubcoreMesh(core_axis_name="c", subcore_axis_name="s")
    deg = pl.kernel(
        _deg_body,
        out_type=jax.ShapeDtypeStruct((NC, NP, DEGW), jnp.float32),
        mesh=mesh,
        scratch_types=[
            pltpu.VMEM((NCHUNK, CHUNK), jnp.int32),
            pltpu.VMEM((CHUNK, DEGW), jnp.float32),
            pltpu.VMEM_SHARED((NP, DEGW), jnp.float32),
        ],
    )
    edge = pl.kernel(
        _edge_body,
        out_type=jax.ShapeDtypeStruct((NC, NP, H), jnp.float32),
        mesh=mesh,
        scratch_types=[
            pltpu.VMEM((NCHUNK, CHUNK), jnp.int32),
            pltpu.VMEM((NCHUNK, CHUNK), jnp.int32),
            pltpu.VMEM((CHUNK, H), jnp.float32),
            pltpu.VMEM_SHARED((NP, H), jnp.float32),
            pltpu.SemaphoreType.DMA,
        ],
    )
    return deg, edge


def _deg_pass(dst3, ones_d, zeros_d):
    return _sc_kernels()[0](dst3, ones_d, zeros_d)


def _edge_pass(m2, src3, dst3, zeros_h):
    return _sc_kernels()[1](m2, src3, dst3, zeros_h)


def _k0_body(x_ref, w_ref, degp_ref, m2_ref, dis_ref):
    deg = 1.0 + degp_ref[0, :, 0:1] + degp_ref[1, :, 0:1]
    dis = lax.rsqrt(deg)
    dis_ref[...] = dis
    m2_ref[...] = jnp.dot(x_ref[...], w_ref[...],
                          preferred_element_type=jnp.float32) * dis


def _layer_body(s_ref, m2p_ref, dis_ref, k1_ref, k2_ref, w_ref, out_ref):
    dis = dis_ref[...]
    t = (s_ref[0] + s_ref[1] + m2p_ref[...]) * dis
    h = jnp.maximum(t * k1_ref[...] + k2_ref[...], 0.0)
    out_ref[...] = jnp.dot(h, w_ref[...],
                           preferred_element_type=jnp.float32) * dis


def _final_body(s_ref, m2p_ref, dis_ref, k1_ref, k2_ref, batch_ref,
                cw1_ref, cb1_ref, cw2_ref, cb2_ref,
                nw1_ref, nb1_ref, nw2_ref, nb2_ref,
                logits_ref, nov_ref, g_acc):
    i = pl.program_id(0)
    t = (s_ref[0] + s_ref[1] + m2p_ref[...]) * dis_ref[...]
    h = jnp.maximum(t * k1_ref[...] + k2_ref[...], 0.0)
    onehot = (batch_ref[...] ==
              lax.broadcasted_iota(jnp.int32, (RB, G), 1)).astype(jnp.float32)
    part = lax.dot_general(onehot, h, (((0,), (0,)), ((), ())),
                           preferred_element_type=jnp.float32)

    @pl.when(i == 0)
    def _():
        g_acc[...] = part

    @pl.when(i > 0)
    def _():
        g_acc[...] = g_acc[...] + part

    @pl.when(i == NRB - 1)
    def _():
        g = g_acc[...]
        t1 = jnp.maximum(jnp.dot(g, cw1_ref[...],
                                 preferred_element_type=jnp.float32)
                         + cb1_ref[...], 0.0)
        logits_ref[...] = jnp.dot(t1, cw2_ref[...],
                                  preferred_element_type=jnp.float32) + cb2_ref[...]
        t2 = jnp.maximum(jnp.dot(g, nw1_ref[...],
                                 preferred_element_type=jnp.float32)
                         + nb1_ref[...], 0.0)
        nov_ref[...] = jax.nn.sigmoid(
            jnp.dot(t2, nw2_ref[...], preferred_element_type=jnp.float32)
            + nb2_ref[...])


_ROW = lambda i: (i, 0)
_CONST2 = lambda i: (0, 0)
_SPLIT = lambda i: (0, i, 0)


def _tc_first(x, w0, degp):
    return pl.pallas_call(
        _k0_body,
        grid=(NRB,),
        in_specs=[
            pl.BlockSpec((RB, D), _ROW),
            pl.BlockSpec((D, H), _CONST2),
            pl.BlockSpec((NC, RB, DEGW), _SPLIT),
        ],
        out_specs=[pl.BlockSpec((RB, H), _ROW), pl.BlockSpec((RB, 1), _ROW)],
        out_shape=[jax.ShapeDtypeStruct((N, H), jnp.float32),
                   jax.ShapeDtypeStruct((N, 1), jnp.float32)],
    )(x, w0, degp)


def _tc_layer(s, m2, dis2d, k1, k2, w):
    return pl.pallas_call(
        _layer_body,
        grid=(NRB,),
        in_specs=[
            pl.BlockSpec((NC, RB, H), _SPLIT),
            pl.BlockSpec((RB, H), _ROW),
            pl.BlockSpec((RB, 1), _ROW),
            pl.BlockSpec((1, H), _CONST2),
            pl.BlockSpec((1, H), _CONST2),
            pl.BlockSpec((H, H), _CONST2),
        ],
        out_specs=pl.BlockSpec((RB, H), _ROW),
        out_shape=jax.ShapeDtypeStruct((N, H), jnp.float32),
    )(s, m2, dis2d, k1, k2, w)


def _tc_final(s, m2, dis2d, k1, k2, batch2,
              cw1, cb1, cw2, cb2, nw1, nb1, nw2, nb2):
    return pl.pallas_call(
        _final_body,
        grid=(NRB,),
        in_specs=[
            pl.BlockSpec((NC, RB, H), _SPLIT),
            pl.BlockSpec((RB, H), _ROW),
            pl.BlockSpec((RB, 1), _ROW),
            pl.BlockSpec((1, H), _CONST2),
            pl.BlockSpec((1, H), _CONST2),
            pl.BlockSpec((RB, 1), _ROW),
            pl.BlockSpec((H, H), _CONST2),
            pl.BlockSpec((1, H), _CONST2),
            pl.BlockSpec((H, C), _CONST2),
            pl.BlockSpec((1, C), _CONST2),
            pl.BlockSpec((H, H), _CONST2),
            pl.BlockSpec((1, H), _CONST2),
            pl.BlockSpec((H, 1), _CONST2),
            pl.BlockSpec((1, 1), _CONST2),
        ],
        out_specs=[pl.BlockSpec((G, C), _CONST2),
                   pl.BlockSpec((G, 1), _CONST2)],
        out_shape=[jax.ShapeDtypeStruct((G, C), jnp.float32),
                   jax.ShapeDtypeStruct((G, 1), jnp.float32)],
        scratch_shapes=[pltpu.VMEM((G, H), jnp.float32)],
    )(s, m2, dis2d, k1, k2, batch2, cw1, cb1, cw2, cb2, nw1, nb1, nw2, nb2)


def kernel(x, edge_index, batch, conv_W, conv_b, bn_w, bn_b,
           cls_W1, cls_b1, cls_W2, cls_b2, nov_W1, nov_b1, nov_W2, nov_b2):
    src_t = edge_index[0].reshape(NW, EPT)
    dst_t = edge_index[1].reshape(NW, EPT)
    pad_i = jnp.arange(NPAD, dtype=jnp.int32)
    tile_i = jnp.arange(NW, dtype=jnp.int32)[:, None]
    src_pad = (tile_i * 311 + pad_i * 89) % N
    dst_pad = N + ((tile_i * 7 + pad_i) % (NP - N))
    src3 = jnp.concatenate([src_t, src_pad], axis=1).reshape(NW, NCHUNK, CHUNK)
    dst3 = jnp.concatenate([dst_t, dst_pad], axis=1).reshape(NW, NCHUNK, CHUNK)
    zeros_h = jnp.zeros((NP, H), jnp.float32)
    zeros_d = jnp.zeros((NP, DEGW), jnp.float32)
    ones_d = jnp.ones((CHUNK, DEGW), jnp.float32)
    batch2 = batch.reshape(N, 1)

    k1 = bn_w * (1.0 / jnp.sqrt(jnp.float32(1.0 + 1e-5)))
    k2 = conv_b * k1 + bn_b

    degp = _deg_pass(dst3, ones_d, zeros_d)
    m2, dis2d = _tc_first(x, conv_W[0], degp)
    for l in range(L - 1):
        s = _edge_pass(m2, src3, dst3, zeros_h)
        m2 = _tc_layer(s, m2, dis2d,
                       k1[l].reshape(1, H), k2[l].reshape(1, H),
                       conv_W[l + 1])
    s = _edge_pass(m2, src3, dst3, zeros_h)
    logits, nov = _tc_final(
        s, m2, dis2d, k1[L - 1].reshape(1, H), k2[L - 1].reshape(1, H),
        batch2,
        cls_W1, cls_b1.reshape(1, H), cls_W2, cls_b2.reshape(1, C),
        nov_W1, nov_b1.reshape(1, H), nov_W2, nov_b2.reshape(1, 1))
    return logits, nov
